# baseline (speedup 1.0000x reference)
import functools

import jax
import jax.numpy as jnp
from jax import lax
from jax.experimental import pallas as pl
from jax.experimental.pallas import tpu as pltpu
from jax.experimental.pallas import tpu_sc as plsc

_D = 64
_S = 4
_NW = 32


def _gather_rows(x, table):
    b, l = x.shape
    seqs_per_w = b // _NW
    rounds = seqs_per_w // _S
    assert seqs_per_w % _S == 0 and rounds % 2 == 0 and rounds >= 4

    mesh = plsc.VectorSubcoreMesh(core_axis_name="c", subcore_axis_name="s")

    @functools.partial(
        pl.kernel,
        mesh=mesh,
        out_type=jax.ShapeDtypeStruct((b, l, _D), jnp.float32),
        scratch_types=[
            pltpu.VMEM((seqs_per_w, l), jnp.int32),
            pltpu.VMEM((_S, l, _D), jnp.float32),
            pltpu.VMEM((_S, l, _D), jnp.float32),
            pltpu.SemaphoreType.DMA,
            pltpu.SemaphoreType.DMA,
            pltpu.SemaphoreType.DMA,
            pltpu.SemaphoreType.DMA,
        ],
        compiler_params=pltpu.CompilerParams(use_tc_tiling_on_sc=False),
    )
    def body(idx_hbm, table_hbm, out_hbm, idx_v, rows0, rows1,
             gsem0, gsem1, ssem0, ssem1):
        wid = lax.axis_index("s") * 2 + lax.axis_index("c")
        seq_base = wid * seqs_per_w
        pltpu.sync_copy(idx_hbm.at[pl.ds(seq_base, seqs_per_w)], idx_v)

        def fire_gathers(r, grp, gsem):
            for s in range(_S):
                pltpu.async_copy(table_hbm.at[idx_v.at[r * _S + s]],
                                 grp.at[s], gsem)

        def wait_gathers(r, grp, gsem):
            for s in range(_S):
                pltpu.make_async_copy(table_hbm.at[idx_v.at[r * _S + s]],
                                      grp.at[s], gsem).wait()

        def fire_scatter(r, grp, ssem):
            pltpu.async_copy(grp, out_hbm.at[pl.ds(seq_base + r * _S, _S)],
                             ssem)

        def wait_scatter(r, grp, ssem):
            pltpu.make_async_copy(grp,
                                  out_hbm.at[pl.ds(seq_base + r * _S, _S)],
                                  ssem).wait()

        fire_gathers(0, rows0, gsem0)
        fire_gathers(1, rows1, gsem1)
        wait_gathers(0, rows0, gsem0)
        fire_scatter(0, rows0, ssem0)

        @pl.loop(0, (rounds - 2) // 2)
        def _steady(i):
            r = 1 + 2 * i
            wait_scatter(r - 1, rows0, ssem0)
            fire_gathers(r + 1, rows0, gsem0)
            wait_gathers(r, rows1, gsem1)
            fire_scatter(r, rows1, ssem1)
            wait_scatter(r, rows1, ssem1)
            fire_gathers(r + 2, rows1, gsem1)
            wait_gathers(r + 1, rows0, gsem0)
            fire_scatter(r + 1, rows0, ssem0)

        r_last = rounds - 1
        wait_scatter(r_last - 1, rows0, ssem0)
        wait_gathers(r_last, rows1, gsem1)
        fire_scatter(r_last, rows1, ssem1)
        wait_scatter(r_last, rows1, ssem1)

    return body(x, table)


_SB = 32


def _unpack_block(i_ref, o_ref):
    nb, l, d = o_ref.shape
    iv = i_ref[...].reshape(nb, l // 2, 2 * d)
    left = iv[:, :, :d]
    right = iv[:, :, d:]
    o_ref[...] = jnp.stack([left, right], axis=2).reshape(nb, l, d)


def _to_native_layout(packed, b, l):
    grid = b // _SB
    rows_per_blk = _SB * l // 2
    return pl.pallas_call(
        _unpack_block,
        grid=(grid,),
        in_specs=[pl.BlockSpec((rows_per_blk, 2 * _D), lambda i: (i, 0))],
        out_specs=pl.BlockSpec((_SB, l, _D), lambda i: (i, 0, 0)),
        out_shape=jax.ShapeDtypeStruct((b, l, _D), jnp.float32),
    )(packed)


def kernel(x, pe):
    b, l = x.shape
    out = _gather_rows(x, pe)
    packed = out.reshape(b * l // 2, 2 * _D)
    return _to_native_layout(packed, b, l)


# speedup vs baseline: 2.2315x; 2.2315x over previous
import functools

import jax
import jax.numpy as jnp
from jax import lax
from jax.experimental import pallas as pl
from jax.experimental.pallas import tpu as pltpu
from jax.experimental.pallas import tpu_sc as plsc

_D = 64
_S = 4
_NW = 32


def _gather_rows(x, table):
    b, l = x.shape
    seqs_per_w = b // _NW
    rounds = seqs_per_w // _S
    assert seqs_per_w % _S == 0 and rounds % 2 == 0 and rounds >= 4

    mesh = plsc.VectorSubcoreMesh(core_axis_name="c", subcore_axis_name="s")

    @functools.partial(
        pl.kernel,
        mesh=mesh,
        out_type=jax.ShapeDtypeStruct((b, l, _D), jnp.float32),
        scratch_types=[
            pltpu.VMEM((seqs_per_w, l), jnp.int32),
            pltpu.VMEM((_S, l, _D), jnp.float32),
            pltpu.VMEM((_S, l, _D), jnp.float32),
            pltpu.SemaphoreType.DMA,
            pltpu.SemaphoreType.DMA,
            pltpu.SemaphoreType.DMA,
            pltpu.SemaphoreType.DMA,
        ],
        compiler_params=pltpu.CompilerParams(use_tc_tiling_on_sc=False),
    )
    def body(idx_hbm, table_hbm, out_hbm, idx_v, rows0, rows1,
             gsem0, gsem1, ssem0, ssem1):
        wid = lax.axis_index("s") * 2 + lax.axis_index("c")
        seq_base = wid * seqs_per_w
        pltpu.sync_copy(idx_hbm.at[pl.ds(seq_base, seqs_per_w)], idx_v)

        def fire_gathers(r, grp, gsem):
            for s in range(_S):
                pltpu.async_copy(table_hbm.at[idx_v.at[r * _S + s]],
                                 grp.at[s], gsem)

        def wait_gathers(r, grp, gsem):
            for s in range(_S):
                pltpu.make_async_copy(table_hbm.at[idx_v.at[r * _S + s]],
                                      grp.at[s], gsem).wait()

        def fire_scatter(r, grp, ssem):
            pltpu.async_copy(grp, out_hbm.at[pl.ds(seq_base + r * _S, _S)],
                             ssem)

        def wait_scatter(r, grp, ssem):
            pltpu.make_async_copy(grp,
                                  out_hbm.at[pl.ds(seq_base + r * _S, _S)],
                                  ssem).wait()

        fire_gathers(0, rows0, gsem0)
        fire_gathers(1, rows1, gsem1)
        wait_gathers(0, rows0, gsem0)
        fire_scatter(0, rows0, ssem0)

        @pl.loop(0, (rounds - 2) // 2)
        def _steady(i):
            r = 1 + 2 * i
            wait_scatter(r - 1, rows0, ssem0)
            fire_gathers(r + 1, rows0, gsem0)
            wait_gathers(r, rows1, gsem1)
            fire_scatter(r, rows1, ssem1)
            wait_scatter(r, rows1, ssem1)
            fire_gathers(r + 2, rows1, gsem1)
            wait_gathers(r + 1, rows0, gsem0)
            fire_scatter(r + 1, rows0, ssem0)

        r_last = rounds - 1
        wait_scatter(r_last - 1, rows0, ssem0)
        wait_gathers(r_last, rows1, gsem1)
        fire_scatter(r_last, rows1, ssem1)
        wait_scatter(r_last, rows1, ssem1)

    return body(x, table)


_BB = 128


def _xpose_block(i_ref, o_ref):
    l, d, bb = o_ref.shape
    iv = i_ref[...].reshape(bb, l // 2, 2 * d)
    for g in range(l // 2):
        a = iv[:, g, :]
        o_ref[2 * g] = a[:, :_D].T
        o_ref[2 * g + 1] = a[:, _D:].T


def _to_entry_layout(packed, b, l):
    return pl.pallas_call(
        _xpose_block,
        grid=(b // _BB,),
        in_specs=[pl.BlockSpec((_BB * l // 2, 2 * _D), lambda j: (j, 0))],
        out_specs=pl.BlockSpec((l, _D, _BB), lambda j: (0, 0, j)),
        out_shape=jax.ShapeDtypeStruct((l, _D, b), jnp.float32),
    )(packed)


def kernel(x, pe):
    b, l = x.shape
    out = _gather_rows(x, pe)
    packed = out.reshape(b * l // 2, 2 * _D)
    return _to_entry_layout(packed, b, l).transpose(2, 0, 1)
